# Initial kernel scaffold; baseline (speedup 1.0000x reference)
import jax
import jax.numpy as jnp
from jax.experimental import pallas as pl


def kernel(x, edge_index, W1, b1, W2, b2):
    raise NotImplementedError("write your pallas kernel here")


# speedup vs baseline: 80.3036x; 80.3036x over previous
import functools

import jax
import jax.numpy as jnp
from jax import lax
from jax.experimental import pallas as pl
from jax.experimental.pallas import tpu as pltpu
from jax.experimental.pallas import tpu_sc as plsc

N = 100000
N_PAD = 100352
PAD_ROWS = N_PAD - N
ROWS = N_PAD // 16
VREGS = ROWS // 16

E = 6400000
CHUNK = 128
INNER = 32
OUTER = 49
PER_TILE = OUTER * INNER * CHUNK
E_PAD = PER_TILE * 32

_MESH = plsc.VectorSubcoreMesh(core_axis_name="c", subcore_axis_name="s")
_SC_PARAMS = pltpu.CompilerParams(needs_layout_passes=False)
_f32 = jnp.float32


def _rsqrt16(d):
    i = plsc.bitcast(d, jnp.int32)
    i = 0x5F3759DF - (i >> 1)
    g = plsc.bitcast(i, _f32)
    for _ in range(3):
        g = g * (1.5 - 0.5 * d * g * g)
    return g


def _row_slice(s):
    return pl.ds(s * ROWS, ROWS)


def _edge_loop(srch, dsth, wid, y0sh, y1sh, a0sh, a1sh, sidx, didx, m0, m1):
    def outer(o, _):
        pltpu.sync_copy(srch.at[wid, pl.ds(o * INNER, INNER)], sidx)
        pltpu.sync_copy(dsth.at[wid, pl.ds(o * INNER, INNER)], didx)

        def inner(j, _):
            pltpu.sync_copy(y0sh.at[sidx.at[j]], m0)
            pltpu.sync_copy(y1sh.at[sidx.at[j]], m1)
            pltpu.sync_copy(m0, a0sh.at[didx.at[j]], add=True)
            pltpu.sync_copy(m1, a1sh.at[didx.at[j]], add=True)
            return 0

        lax.fori_loop(0, INNER, inner, 0)
        return 0

    lax.fori_loop(0, OUTER, outer, 0)


def _deg_body(dsth, deg_out, acc, zbuf, ones, idx):
    c = lax.axis_index("c")
    s = lax.axis_index("s")
    wid = c * 16 + s
    sl = _row_slice(s)

    def fill(i, _):
        zbuf[pl.ds(i * 16, 16)] = jnp.zeros((16,), _f32)
        return 0

    lax.fori_loop(0, VREGS, fill, 0)
    for i in range(CHUNK // 16):
        ones[pl.ds(i * 16, 16)] = jnp.ones((16,), _f32)
    pltpu.sync_copy(zbuf, acc.at[sl])
    plsc.subcore_barrier()

    def outer(o, _):
        pltpu.sync_copy(dsth.at[wid, pl.ds(o * INNER, INNER)], idx)

        def inner(j, _):
            pltpu.sync_copy(ones, acc.at[idx.at[j]], add=True)
            return 0

        lax.fori_loop(0, INNER, inner, 0)
        return 0

    lax.fori_loop(0, OUTER, outer, 0)
    plsc.subcore_barrier()
    pltpu.sync_copy(acc.at[sl], deg_out.at[c, sl])


_deg_call = functools.partial(
    pl.kernel,
    out_type=jax.ShapeDtypeStruct((2, N_PAD), _f32),
    mesh=_MESH,
    compiler_params=_SC_PARAMS,
    scratch_types=[
        pltpu.VMEM_SHARED((N_PAD,), _f32),
        pltpu.VMEM((ROWS,), _f32),
        pltpu.VMEM((CHUNK,), _f32),
        pltpu.VMEM((INNER, CHUNK), jnp.int32),
    ],
)(_deg_body)


def _agg1_body(degp, x0h, x1h, srch, dsth, aggp_out, s_out,
               y0sh, y1sh, a0sh, a1sh,
               d0, d1, x0v, x1v, sv, y0v, y1v, z0v, z1v,
               sidx, didx, m0, m1):
    c = lax.axis_index("c")
    s = lax.axis_index("s")
    wid = c * 16 + s
    sl = _row_slice(s)

    pltpu.sync_copy(degp.at[0, sl], d0)
    pltpu.sync_copy(degp.at[1, sl], d1)
    pltpu.sync_copy(x0h.at[sl], x0v)
    pltpu.sync_copy(x1h.at[sl], x1v)
    fsel = jnp.where(c == 0, 1.0, 0.0).astype(_f32)

    def ew(i, _):
        ds_ = pl.ds(i * 16, 16)
        d = d0[ds_] + d1[ds_] + 1.0
        g = _rsqrt16(d)
        sv[ds_] = g
        y0 = x0v[ds_] * g
        y1 = x1v[ds_] * g
        y0v[ds_] = y0
        y1v[ds_] = y1
        z0v[ds_] = y0 * fsel
        z1v[ds_] = y1 * fsel
        return 0

    lax.fori_loop(0, VREGS, ew, 0)
    pltpu.sync_copy(y0v, y0sh.at[sl])
    pltpu.sync_copy(y1v, y1sh.at[sl])
    pltpu.sync_copy(z0v, a0sh.at[sl])
    pltpu.sync_copy(z1v, a1sh.at[sl])

    @pl.when(c == 0)
    def _():
        pltpu.sync_copy(sv, s_out.at[sl])

    plsc.subcore_barrier()
    _edge_loop(srch, dsth, wid, y0sh, y1sh, a0sh, a1sh, sidx, didx, m0, m1)
    plsc.subcore_barrier()
    pltpu.sync_copy(a0sh.at[sl], aggp_out.at[c, 0, sl])
    pltpu.sync_copy(a1sh.at[sl], aggp_out.at[c, 1, sl])


_agg1_call = functools.partial(
    pl.kernel,
    out_type=(jax.ShapeDtypeStruct((2, 2, N_PAD), _f32),
              jax.ShapeDtypeStruct((N_PAD,), _f32)),
    mesh=_MESH,
    compiler_params=_SC_PARAMS,
    scratch_types=[
        pltpu.VMEM_SHARED((N_PAD,), _f32),
        pltpu.VMEM_SHARED((N_PAD,), _f32),
        pltpu.VMEM_SHARED((N_PAD,), _f32),
        pltpu.VMEM_SHARED((N_PAD,), _f32),
        pltpu.VMEM((ROWS,), _f32),
        pltpu.VMEM((ROWS,), _f32),
        pltpu.VMEM((ROWS,), _f32),
        pltpu.VMEM((ROWS,), _f32),
        pltpu.VMEM((ROWS,), _f32),
        pltpu.VMEM((ROWS,), _f32),
        pltpu.VMEM((ROWS,), _f32),
        pltpu.VMEM((ROWS,), _f32),
        pltpu.VMEM((ROWS,), _f32),
        pltpu.VMEM((INNER, CHUNK), jnp.int32),
        pltpu.VMEM((INNER, CHUNK), jnp.int32),
        pltpu.VMEM((CHUNK,), _f32),
        pltpu.VMEM((CHUNK,), _f32),
    ],
)(_agg1_body)


def _agg2_body(aggp1, sh, w1h, b1h, w2h, srch, dsth, aggp_out,
               y0sh, y1sh, a0sh, a1sh,
               p00, p01, p10, p11, sv, y0v, y1v, z0v, z1v,
               wv1, bv1, wv2, sidx, didx, m0, m1):
    c = lax.axis_index("c")
    s = lax.axis_index("s")
    wid = c * 16 + s
    sl = _row_slice(s)

    pltpu.sync_copy(aggp1.at[0, 0, sl], p00)
    pltpu.sync_copy(aggp1.at[0, 1, sl], p01)
    pltpu.sync_copy(aggp1.at[1, 0, sl], p10)
    pltpu.sync_copy(aggp1.at[1, 1, sl], p11)
    pltpu.sync_copy(sh.at[sl], sv)
    pltpu.sync_copy(w1h, wv1)
    pltpu.sync_copy(b1h, bv1)
    pltpu.sync_copy(w2h, wv2)
    w1vec = wv1[...]
    b1vec = bv1[...]
    w2vec = wv2[...]
    w1c = [w1vec[j] for j in range(16)]
    b1c = [b1vec[j] for j in range(8)]
    w2c = [w2vec[j] for j in range(16)]
    fsel = jnp.where(c == 0, 1.0, 0.0).astype(_f32)

    def ew(i, _):
        ds_ = pl.ds(i * 16, 16)
        g = sv[ds_]
        a0 = (p00[ds_] + p10[ds_]) * g
        a1 = (p01[ds_] + p11[ds_]) * g
        q0 = jnp.zeros((16,), _f32)
        q1 = jnp.zeros((16,), _f32)
        for j in range(8):
            h = jnp.maximum(a0 * w1c[j] + a1 * w1c[8 + j] + b1c[j], 0.0)
            q0 = q0 + h * w2c[2 * j]
            q1 = q1 + h * w2c[2 * j + 1]
        y0 = q0 * g
        y1 = q1 * g
        y0v[ds_] = y0
        y1v[ds_] = y1
        z0v[ds_] = y0 * fsel
        z1v[ds_] = y1 * fsel
        return 0

    lax.fori_loop(0, VREGS, ew, 0)
    pltpu.sync_copy(y0v, y0sh.at[sl])
    pltpu.sync_copy(y1v, y1sh.at[sl])
    pltpu.sync_copy(z0v, a0sh.at[sl])
    pltpu.sync_copy(z1v, a1sh.at[sl])
    plsc.subcore_barrier()
    _edge_loop(srch, dsth, wid, y0sh, y1sh, a0sh, a1sh, sidx, didx, m0, m1)
    plsc.subcore_barrier()
    pltpu.sync_copy(a0sh.at[sl], aggp_out.at[c, 0, sl])
    pltpu.sync_copy(a1sh.at[sl], aggp_out.at[c, 1, sl])


_agg2_call = functools.partial(
    pl.kernel,
    out_type=jax.ShapeDtypeStruct((2, 2, N_PAD), _f32),
    mesh=_MESH,
    compiler_params=_SC_PARAMS,
    scratch_types=[
        pltpu.VMEM_SHARED((N_PAD,), _f32),
        pltpu.VMEM_SHARED((N_PAD,), _f32),
        pltpu.VMEM_SHARED((N_PAD,), _f32),
        pltpu.VMEM_SHARED((N_PAD,), _f32),
        pltpu.VMEM((ROWS,), _f32),
        pltpu.VMEM((ROWS,), _f32),
        pltpu.VMEM((ROWS,), _f32),
        pltpu.VMEM((ROWS,), _f32),
        pltpu.VMEM((ROWS,), _f32),
        pltpu.VMEM((ROWS,), _f32),
        pltpu.VMEM((ROWS,), _f32),
        pltpu.VMEM((ROWS,), _f32),
        pltpu.VMEM((ROWS,), _f32),
        pltpu.VMEM((16,), _f32),
        pltpu.VMEM((16,), _f32),
        pltpu.VMEM((16,), _f32),
        pltpu.VMEM((INNER, CHUNK), jnp.int32),
        pltpu.VMEM((INNER, CHUNK), jnp.int32),
        pltpu.VMEM((CHUNK,), _f32),
        pltpu.VMEM((CHUNK,), _f32),
    ],
)(_agg2_body)


def _fin_body(aggp_ref, s_ref, b2_ref, o_ref):
    sv = s_ref[...]
    for f in range(2):
        o_ref[f] = (aggp_ref[0, f] + aggp_ref[1, f]) * sv + b2_ref[f, 0:1, :]


_fin_call = pl.pallas_call(
    _fin_body,
    out_shape=jax.ShapeDtypeStruct((2, 784, 128), _f32),
)


def kernel(x, edge_index, W1, b1, W2, b2):
    x = x.astype(_f32)
    x0 = jnp.pad(x[:, 0], (0, PAD_ROWS))
    x1 = jnp.pad(x[:, 1], (0, PAD_ROWS))
    npad = E_PAD - E
    fill = N + (jnp.arange(npad, dtype=jnp.int32) % PAD_ROWS)
    src = jnp.concatenate([edge_index[0].astype(jnp.int32), fill])
    dst = jnp.concatenate([edge_index[1].astype(jnp.int32), fill])
    src3 = src.reshape(32, OUTER * INNER, CHUNK)
    dst3 = dst.reshape(32, OUTER * INNER, CHUNK)

    degp = _deg_call(dst3)
    aggp1, svec = _agg1_call(degp, x0, x1, src3, dst3)
    aggp2 = _agg2_call(
        aggp1, svec,
        W1.astype(_f32).reshape(16),
        jnp.pad(b1.astype(_f32), (0, 8)),
        W2.astype(_f32).reshape(16),
        src3, dst3)

    out = _fin_call(
        aggp2.reshape(2, 2, 784, 128),
        svec.reshape(784, 128),
        jnp.broadcast_to(b2.astype(_f32).reshape(2, 1, 1), (2, 1, 128)))
    out = out.reshape(2, N_PAD)
    return jnp.stack([out[0, :N], out[1, :N]], axis=1)


# speedup vs baseline: 118.9347x; 1.4811x over previous
import functools

import jax
import jax.numpy as jnp
from jax import lax
from jax.experimental import pallas as pl
from jax.experimental.pallas import tpu as pltpu
from jax.experimental.pallas import tpu_sc as plsc

N = 100000
N_PAD = 100352
PAD_ROWS = N_PAD - N
ROWS = N_PAD // 16

E = 6400000
E_SELF = E + N
CHUNK = 128
INNER = 8
OUTER = 200
PER_TILE = OUTER * INNER * CHUNK
E_PAD = PER_TILE * 32

_MESH = plsc.VectorSubcoreMesh(core_axis_name="c", subcore_axis_name="s")
_SC_PARAMS = pltpu.CompilerParams(needs_layout_passes=False,
                                  use_tc_tiling_on_sc=False)
_f32 = jnp.float32


def _row_slice(s):
    return pl.ds(s * ROWS, ROWS)


def _fire_gathers(t0, t1, sidx, m0, m1, gsem):
    def fire(j, _):
        pltpu.async_copy(t0.at[sidx.at[j]], m0.at[j], gsem)
        pltpu.async_copy(t1.at[sidx.at[j]], m1.at[j], gsem)
        return 0

    lax.fori_loop(0, INNER, fire, 0)


def _drain_gathers(t0, t1, sidx, m0, m1, gsem):
    def drain(j, _):
        pltpu.make_async_copy(t0.at[sidx.at[j]], m0.at[j], gsem).wait()
        pltpu.make_async_copy(t1.at[sidx.at[j]], m1.at[j], gsem).wait()
        return 0

    lax.fori_loop(0, INNER, drain, 0)


def _fire_scatters(m0, m1, didx, a0, a1, ssem):
    def fire(j, _):
        pltpu.async_copy(m0.at[j], a0.at[didx.at[j]], ssem, add=True)
        pltpu.async_copy(m1.at[j], a1.at[didx.at[j]], ssem, add=True)
        return 0

    lax.fori_loop(0, INNER, fire, 0)


def _drain_scatters(m0, m1, didx, a0, a1, ssem):
    def drain(j, _):
        pltpu.make_async_copy(m0.at[j], a0.at[didx.at[j]], ssem).wait()
        pltpu.make_async_copy(m1.at[j], a1.at[didx.at[j]], ssem).wait()
        return 0

    lax.fori_loop(0, INNER, drain, 0)


def _edge_loop(srch, dsth, wid, t0, t1, a0, a1,
               sidxA, didxA, sidxB, didxB, mA0, mA1, mB0, mB1,
               isem, gsem, ssem):
    pltpu.async_copy(srch.at[wid, pl.ds(0, INNER)], sidxA, isem)
    pltpu.async_copy(dsth.at[wid, pl.ds(0, INNER)], didxA, isem)

    def body(t, _):
        oA = 2 * t
        oB = 2 * t + 1
        slA = pl.ds(oA * INNER, INNER)
        slB = pl.ds(oB * INNER, INNER)
        pltpu.make_async_copy(srch.at[wid, slA], sidxA, isem).wait()
        pltpu.make_async_copy(dsth.at[wid, slA], didxA, isem).wait()
        pltpu.async_copy(srch.at[wid, slB], sidxB, isem)
        pltpu.async_copy(dsth.at[wid, slB], didxB, isem)
        _fire_gathers(t0, t1, sidxA, mA0, mA1, gsem)
        _drain_gathers(t0, t1, sidxA, mA0, mA1, gsem)
        _fire_scatters(mA0, mA1, didxA, a0, a1, ssem)
        pltpu.make_async_copy(srch.at[wid, slB], sidxB, isem).wait()
        pltpu.make_async_copy(dsth.at[wid, slB], didxB, isem).wait()
        _fire_gathers(t0, t1, sidxB, mB0, mB1, gsem)
        _drain_scatters(mA0, mA1, didxA, a0, a1, ssem)

        @pl.when(oB + 1 < OUTER)
        def _():
            sl_n = pl.ds((oB + 1) * INNER, INNER)
            pltpu.async_copy(srch.at[wid, sl_n], sidxA, isem)
            pltpu.async_copy(dsth.at[wid, sl_n], didxA, isem)

        _drain_gathers(t0, t1, sidxB, mB0, mB1, gsem)
        _fire_scatters(mB0, mB1, didxB, a0, a1, ssem)
        _drain_scatters(mB0, mB1, didxB, a0, a1, ssem)
        return 0

    lax.fori_loop(0, OUTER // 2, body, 0)


def _deg_body(dsth, deg_out, acc, zbuf, ones, didxA, didxB, isem, ssem):
    c = lax.axis_index("c")
    s = lax.axis_index("s")
    wid = c * 16 + s
    sl = _row_slice(s)

    def fill(i, _):
        zbuf[pl.ds(i * 16, 16)] = jnp.zeros((16,), _f32)
        return 0

    lax.fori_loop(0, ROWS // 16, fill, 0)
    for i in range(CHUNK // 16):
        ones[pl.ds(i * 16, 16)] = jnp.ones((16,), _f32)
    pltpu.sync_copy(zbuf, acc.at[sl])
    plsc.subcore_barrier()

    pltpu.async_copy(dsth.at[wid, pl.ds(0, INNER)], didxA, isem)

    def fire_ones(didx):
        def fire(j, _):
            pltpu.async_copy(ones, acc.at[didx.at[j]], ssem, add=True)
            return 0

        lax.fori_loop(0, INNER, fire, 0)

    def drain_ones(didx):
        def drain(j, _):
            pltpu.make_async_copy(ones, acc.at[didx.at[j]], ssem).wait()
            return 0

        lax.fori_loop(0, INNER, drain, 0)

    def body(t, _):
        oA = 2 * t
        oB = 2 * t + 1
        pltpu.make_async_copy(dsth.at[wid, pl.ds(oA * INNER, INNER)], didxA,
                              isem).wait()
        pltpu.async_copy(dsth.at[wid, pl.ds(oB * INNER, INNER)], didxB, isem)
        fire_ones(didxA)
        pltpu.make_async_copy(dsth.at[wid, pl.ds(oB * INNER, INNER)], didxB,
                              isem).wait()
        drain_ones(didxA)

        @pl.when(oB + 1 < OUTER)
        def _():
            pltpu.async_copy(dsth.at[wid, pl.ds((oB + 1) * INNER, INNER)],
                             didxA, isem)

        fire_ones(didxB)
        drain_ones(didxB)
        return 0

    lax.fori_loop(0, OUTER // 2, body, 0)
    plsc.subcore_barrier()
    pltpu.sync_copy(acc.at[sl], deg_out.at[c, sl])


_deg_call = functools.partial(
    pl.kernel,
    out_type=jax.ShapeDtypeStruct((2, N_PAD), _f32),
    mesh=_MESH,
    compiler_params=_SC_PARAMS,
    scratch_types=[
        pltpu.VMEM_SHARED((N_PAD,), _f32),
        pltpu.VMEM((ROWS,), _f32),
        pltpu.VMEM((CHUNK,), _f32),
        pltpu.VMEM((INNER, CHUNK), jnp.int32),
        pltpu.VMEM((INNER, CHUNK), jnp.int32),
        pltpu.SemaphoreType.DMA,
        pltpu.SemaphoreType.DMA,
    ],
)(_deg_body)


def _agg_body(y0h, y1h, zh, srch, dsth, aggp_out,
              t0, t1, a0, a1,
              sidxA, didxA, sidxB, didxB, mA0, mA1, mB0, mB1,
              isem, gsem, ssem):
    c = lax.axis_index("c")
    s = lax.axis_index("s")
    wid = c * 16 + s
    sl = _row_slice(s)

    pltpu.sync_copy(y0h.at[sl], t0.at[sl])
    pltpu.sync_copy(y1h.at[sl], t1.at[sl])
    pltpu.sync_copy(zh.at[sl], a0.at[sl])
    pltpu.sync_copy(zh.at[sl], a1.at[sl])
    plsc.subcore_barrier()
    _edge_loop(srch, dsth, wid, t0, t1, a0, a1,
               sidxA, didxA, sidxB, didxB, mA0, mA1, mB0, mB1,
               isem, gsem, ssem)
    plsc.subcore_barrier()
    pltpu.sync_copy(a0.at[sl], aggp_out.at[c, 0, sl])
    pltpu.sync_copy(a1.at[sl], aggp_out.at[c, 1, sl])


_agg_call = functools.partial(
    pl.kernel,
    out_type=jax.ShapeDtypeStruct((2, 2, N_PAD), _f32),
    mesh=_MESH,
    compiler_params=_SC_PARAMS,
    scratch_types=[
        pltpu.VMEM_SHARED((N_PAD,), _f32),
        pltpu.VMEM_SHARED((N_PAD,), _f32),
        pltpu.VMEM_SHARED((N_PAD,), _f32),
        pltpu.VMEM_SHARED((N_PAD,), _f32),
        pltpu.VMEM((INNER, CHUNK), jnp.int32),
        pltpu.VMEM((INNER, CHUNK), jnp.int32),
        pltpu.VMEM((INNER, CHUNK), jnp.int32),
        pltpu.VMEM((INNER, CHUNK), jnp.int32),
        pltpu.VMEM((INNER, CHUNK), _f32),
        pltpu.VMEM((INNER, CHUNK), _f32),
        pltpu.VMEM((INNER, CHUNK), _f32),
        pltpu.VMEM((INNER, CHUNK), _f32),
        pltpu.SemaphoreType.DMA,
        pltpu.SemaphoreType.DMA,
        pltpu.SemaphoreType.DMA,
    ],
)(_agg_body)


def _dense1_body(degp_ref, x_ref, s_ref, y_ref):
    d = degp_ref[0] + degp_ref[1]
    sv = lax.rsqrt(jnp.maximum(d, 1.0))
    s_ref[...] = sv
    y_ref[0] = x_ref[0] * sv
    y_ref[1] = x_ref[1] * sv


_dense1_call = pl.pallas_call(
    _dense1_body,
    out_shape=(jax.ShapeDtypeStruct((784, 128), _f32),
               jax.ShapeDtypeStruct((2, 784, 128), _f32)),
)


def _dense2_body(a_ref, s_ref, w1_ref, b1_ref, w2_ref, y_ref):
    sv = s_ref[...]
    a0 = (a_ref[0, 0] + a_ref[1, 0]) * sv
    a1 = (a_ref[0, 1] + a_ref[1, 1]) * sv
    q0 = jnp.zeros((784, 128), _f32)
    q1 = jnp.zeros((784, 128), _f32)
    for j in range(8):
        h = jnp.maximum(a0 * w1_ref[0, j] + a1 * w1_ref[1, j] + b1_ref[j],
                        0.0)
        q0 = q0 + h * w2_ref[j, 0]
        q1 = q1 + h * w2_ref[j, 1]
    y_ref[0] = q0 * sv
    y_ref[1] = q1 * sv


_dense2_call = pl.pallas_call(
    _dense2_body,
    in_specs=[
        pl.BlockSpec(),
        pl.BlockSpec(),
        pl.BlockSpec(memory_space=pltpu.SMEM),
        pl.BlockSpec(memory_space=pltpu.SMEM),
        pl.BlockSpec(memory_space=pltpu.SMEM),
    ],
    out_shape=jax.ShapeDtypeStruct((2, 784, 128), _f32),
)


def _fin_body(aggp_ref, s_ref, b2_ref, o_ref):
    sv = s_ref[...]
    for f in range(2):
        o_ref[f] = ((aggp_ref[0, f] + aggp_ref[1, f]) * sv
                    + b2_ref[f, 0:1, :])


_fin_call = pl.pallas_call(
    _fin_body,
    out_shape=jax.ShapeDtypeStruct((2, 784, 128), _f32),
)


def kernel(x, edge_index, W1, b1, W2, b2):
    x = x.astype(_f32)
    xp = jnp.pad(x, ((0, PAD_ROWS), (0, 0)))
    loop = jnp.arange(N, dtype=jnp.int32)
    npad = E_PAD - E_SELF
    fillpad = N + (jnp.arange(npad, dtype=jnp.int32) % PAD_ROWS)
    src = jnp.concatenate([edge_index[0].astype(jnp.int32), loop, fillpad])
    dst = jnp.concatenate([edge_index[1].astype(jnp.int32), loop, fillpad])
    src3 = src.reshape(32, OUTER * INNER, CHUNK)
    dst3 = dst.reshape(32, OUTER * INNER, CHUNK)
    zvec = jnp.zeros((N_PAD,), _f32)

    degp = _deg_call(dst3)
    svec, y1p = _dense1_call(degp.reshape(2, 784, 128),
                             xp.T.reshape(2, 784, 128))
    y1f = y1p.reshape(2, N_PAD)
    aggp1 = _agg_call(y1f[0], y1f[1], zvec, src3, dst3)

    y2p = _dense2_call(aggp1.reshape(2, 2, 784, 128), svec,
                       W1.astype(_f32), b1.astype(_f32), W2.astype(_f32))
    y2f = y2p.reshape(2, N_PAD)
    aggp2 = _agg_call(y2f[0], y2f[1], zvec, src3, dst3)

    b2b = jnp.broadcast_to(b2.astype(_f32).reshape(2, 1, 1), (2, 1, 128))
    out = _fin_call(aggp2.reshape(2, 2, 784, 128),
                    svec, b2b)
    return jnp.stack([out[0].reshape(N_PAD), out[1].reshape(N_PAD)],
                     axis=1)[:N]


# speedup vs baseline: 126.9439x; 1.0673x over previous
import functools

import jax
import jax.numpy as jnp
from jax import lax
from jax.experimental import pallas as pl
from jax.experimental.pallas import tpu as pltpu
from jax.experimental.pallas import tpu_sc as plsc

N = 100000
N_PAD = 100352
PAD_ROWS = N_PAD - N
ROWS = N_PAD // 16

E = 6400000
E_SELF = E + N
CHUNK = 128
INNER = 16
OUTER = 100
PER_TILE = OUTER * INNER * CHUNK
E_PAD = PER_TILE * 32

_MESH = plsc.VectorSubcoreMesh(core_axis_name="c", subcore_axis_name="s")
_SC_PARAMS = pltpu.CompilerParams(needs_layout_passes=False,
                                  use_tc_tiling_on_sc=False)
_f32 = jnp.float32


def _row_slice(s):
    return pl.ds(s * ROWS, ROWS)


def _fire_gathers(t0, t1, sidx, m0, m1, gsem):
    def fire(j, _):
        pltpu.async_copy(t0.at[sidx.at[j]], m0.at[j], gsem)
        pltpu.async_copy(t1.at[sidx.at[j]], m1.at[j], gsem)
        return 0

    lax.fori_loop(0, INNER, fire, 0)


def _drain_gathers(t0, t1, sidx, m0, m1, gsem):
    def drain(j, _):
        pltpu.make_async_copy(t0.at[sidx.at[j]], m0.at[j], gsem).wait()
        pltpu.make_async_copy(t1.at[sidx.at[j]], m1.at[j], gsem).wait()
        return 0

    lax.fori_loop(0, INNER, drain, 0)


def _fire_scatters(m0, m1, didx, a0, a1, ssem):
    def fire(j, _):
        pltpu.async_copy(m0.at[j], a0.at[didx.at[j]], ssem, add=True)
        pltpu.async_copy(m1.at[j], a1.at[didx.at[j]], ssem, add=True)
        return 0

    lax.fori_loop(0, INNER, fire, 0)


def _drain_scatters(m0, m1, didx, a0, a1, ssem):
    def drain(j, _):
        pltpu.make_async_copy(m0.at[j], a0.at[didx.at[j]], ssem).wait()
        pltpu.make_async_copy(m1.at[j], a1.at[didx.at[j]], ssem).wait()
        return 0

    lax.fori_loop(0, INNER, drain, 0)


def _edge_loop(srch, dsth, wid, t0, t1, a0, a1,
               sidxA, didxA, sidxB, didxB, mA0, mA1, mB0, mB1,
               isem, gsem, ssem):
    pltpu.async_copy(srch.at[wid, pl.ds(0, INNER)], sidxA, isem)
    pltpu.async_copy(dsth.at[wid, pl.ds(0, INNER)], didxA, isem)

    def body(t, _):
        oA = 2 * t
        oB = 2 * t + 1
        slA = pl.ds(oA * INNER, INNER)
        slB = pl.ds(oB * INNER, INNER)
        pltpu.make_async_copy(srch.at[wid, slA], sidxA, isem).wait()
        pltpu.make_async_copy(dsth.at[wid, slA], didxA, isem).wait()
        pltpu.async_copy(srch.at[wid, slB], sidxB, isem)
        pltpu.async_copy(dsth.at[wid, slB], didxB, isem)
        _fire_gathers(t0, t1, sidxA, mA0, mA1, gsem)
        _drain_gathers(t0, t1, sidxA, mA0, mA1, gsem)
        _fire_scatters(mA0, mA1, didxA, a0, a1, ssem)
        pltpu.make_async_copy(srch.at[wid, slB], sidxB, isem).wait()
        pltpu.make_async_copy(dsth.at[wid, slB], didxB, isem).wait()
        _fire_gathers(t0, t1, sidxB, mB0, mB1, gsem)
        _drain_scatters(mA0, mA1, didxA, a0, a1, ssem)

        @pl.when(oB + 1 < OUTER)
        def _():
            sl_n = pl.ds((oB + 1) * INNER, INNER)
            pltpu.async_copy(srch.at[wid, sl_n], sidxA, isem)
            pltpu.async_copy(dsth.at[wid, sl_n], didxA, isem)

        _drain_gathers(t0, t1, sidxB, mB0, mB1, gsem)
        _fire_scatters(mB0, mB1, didxB, a0, a1, ssem)
        _drain_scatters(mB0, mB1, didxB, a0, a1, ssem)
        return 0

    lax.fori_loop(0, OUTER // 2, body, 0)


def _deg_body(dsth, deg_out, acc, zbuf, ones, didxA, didxB, isem, ssem):
    c = lax.axis_index("c")
    s = lax.axis_index("s")
    wid = c * 16 + s
    sl = _row_slice(s)

    def fill(i, _):
        zbuf[pl.ds(i * 16, 16)] = jnp.zeros((16,), _f32)
        return 0

    lax.fori_loop(0, ROWS // 16, fill, 0)
    for i in range(CHUNK // 16):
        ones[pl.ds(i * 16, 16)] = jnp.ones((16,), _f32)
    pltpu.sync_copy(zbuf, acc.at[sl])
    plsc.subcore_barrier()

    pltpu.async_copy(dsth.at[wid, pl.ds(0, INNER)], didxA, isem)

    def fire_ones(didx):
        def fire(j, _):
            pltpu.async_copy(ones, acc.at[didx.at[j]], ssem, add=True)
            return 0

        lax.fori_loop(0, INNER, fire, 0)

    def drain_ones(didx):
        def drain(j, _):
            pltpu.make_async_copy(ones, acc.at[didx.at[j]], ssem).wait()
            return 0

        lax.fori_loop(0, INNER, drain, 0)

    def body(t, _):
        oA = 2 * t
        oB = 2 * t + 1
        pltpu.make_async_copy(dsth.at[wid, pl.ds(oA * INNER, INNER)], didxA,
                              isem).wait()
        pltpu.async_copy(dsth.at[wid, pl.ds(oB * INNER, INNER)], didxB, isem)
        fire_ones(didxA)
        pltpu.make_async_copy(dsth.at[wid, pl.ds(oB * INNER, INNER)], didxB,
                              isem).wait()
        drain_ones(didxA)

        @pl.when(oB + 1 < OUTER)
        def _():
            pltpu.async_copy(dsth.at[wid, pl.ds((oB + 1) * INNER, INNER)],
                             didxA, isem)

        fire_ones(didxB)
        drain_ones(didxB)
        return 0

    lax.fori_loop(0, OUTER // 2, body, 0)
    plsc.subcore_barrier()
    pltpu.sync_copy(acc.at[sl], deg_out.at[c, sl])


_deg_call = functools.partial(
    pl.kernel,
    out_type=jax.ShapeDtypeStruct((2, N_PAD), _f32),
    mesh=_MESH,
    compiler_params=_SC_PARAMS,
    scratch_types=[
        pltpu.VMEM_SHARED((N_PAD,), _f32),
        pltpu.VMEM((ROWS,), _f32),
        pltpu.VMEM((CHUNK,), _f32),
        pltpu.VMEM((INNER, CHUNK), jnp.int32),
        pltpu.VMEM((INNER, CHUNK), jnp.int32),
        pltpu.SemaphoreType.DMA,
        pltpu.SemaphoreType.DMA,
    ],
)(_deg_body)


def _agg_body(y0h, y1h, zh, srch, dsth, aggp_out,
              t0, t1, a0, a1,
              sidxA, didxA, sidxB, didxB, mA0, mA1, mB0, mB1,
              isem, gsem, ssem):
    c = lax.axis_index("c")
    s = lax.axis_index("s")
    wid = c * 16 + s
    sl = _row_slice(s)

    pltpu.sync_copy(y0h.at[sl], t0.at[sl])
    pltpu.sync_copy(y1h.at[sl], t1.at[sl])
    pltpu.sync_copy(zh.at[sl], a0.at[sl])
    pltpu.sync_copy(zh.at[sl], a1.at[sl])
    plsc.subcore_barrier()
    _edge_loop(srch, dsth, wid, t0, t1, a0, a1,
               sidxA, didxA, sidxB, didxB, mA0, mA1, mB0, mB1,
               isem, gsem, ssem)
    plsc.subcore_barrier()
    pltpu.sync_copy(a0.at[sl], aggp_out.at[c, 0, sl])
    pltpu.sync_copy(a1.at[sl], aggp_out.at[c, 1, sl])


_agg_call = functools.partial(
    pl.kernel,
    out_type=jax.ShapeDtypeStruct((2, 2, N_PAD), _f32),
    mesh=_MESH,
    compiler_params=_SC_PARAMS,
    scratch_types=[
        pltpu.VMEM_SHARED((N_PAD,), _f32),
        pltpu.VMEM_SHARED((N_PAD,), _f32),
        pltpu.VMEM_SHARED((N_PAD,), _f32),
        pltpu.VMEM_SHARED((N_PAD,), _f32),
        pltpu.VMEM((INNER, CHUNK), jnp.int32),
        pltpu.VMEM((INNER, CHUNK), jnp.int32),
        pltpu.VMEM((INNER, CHUNK), jnp.int32),
        pltpu.VMEM((INNER, CHUNK), jnp.int32),
        pltpu.VMEM((INNER, CHUNK), _f32),
        pltpu.VMEM((INNER, CHUNK), _f32),
        pltpu.VMEM((INNER, CHUNK), _f32),
        pltpu.VMEM((INNER, CHUNK), _f32),
        pltpu.SemaphoreType.DMA,
        pltpu.SemaphoreType.DMA,
        pltpu.SemaphoreType.DMA,
    ],
)(_agg_body)


def _dense1_body(degp_ref, x_ref, s_ref, y_ref):
    d = degp_ref[0] + degp_ref[1]
    sv = lax.rsqrt(jnp.maximum(d, 1.0))
    s_ref[...] = sv
    y_ref[0] = x_ref[0] * sv
    y_ref[1] = x_ref[1] * sv


_dense1_call = pl.pallas_call(
    _dense1_body,
    out_shape=(jax.ShapeDtypeStruct((784, 128), _f32),
               jax.ShapeDtypeStruct((2, 784, 128), _f32)),
)


def _dense2_body(a_ref, s_ref, w1_ref, b1_ref, w2_ref, y_ref):
    sv = s_ref[...]
    a0 = (a_ref[0, 0] + a_ref[1, 0]) * sv
    a1 = (a_ref[0, 1] + a_ref[1, 1]) * sv
    q0 = jnp.zeros((784, 128), _f32)
    q1 = jnp.zeros((784, 128), _f32)
    for j in range(8):
        h = jnp.maximum(a0 * w1_ref[0, j] + a1 * w1_ref[1, j] + b1_ref[j],
                        0.0)
        q0 = q0 + h * w2_ref[j, 0]
        q1 = q1 + h * w2_ref[j, 1]
    y_ref[0] = q0 * sv
    y_ref[1] = q1 * sv


_dense2_call = pl.pallas_call(
    _dense2_body,
    in_specs=[
        pl.BlockSpec(),
        pl.BlockSpec(),
        pl.BlockSpec(memory_space=pltpu.SMEM),
        pl.BlockSpec(memory_space=pltpu.SMEM),
        pl.BlockSpec(memory_space=pltpu.SMEM),
    ],
    out_shape=jax.ShapeDtypeStruct((2, 784, 128), _f32),
)


def _fin_body(aggp_ref, s_ref, b2_ref, o_ref):
    sv = s_ref[...]
    for f in range(2):
        o_ref[f] = ((aggp_ref[0, f] + aggp_ref[1, f]) * sv
                    + b2_ref[f, 0:1, :])


_fin_call = pl.pallas_call(
    _fin_body,
    out_shape=jax.ShapeDtypeStruct((2, 784, 128), _f32),
)


def kernel(x, edge_index, W1, b1, W2, b2):
    x = x.astype(_f32)
    xp = jnp.pad(x, ((0, PAD_ROWS), (0, 0)))
    loop = jnp.arange(N, dtype=jnp.int32)
    npad = E_PAD - E_SELF
    fillpad = N + (jnp.arange(npad, dtype=jnp.int32) % PAD_ROWS)
    src = jnp.concatenate([edge_index[0].astype(jnp.int32), loop, fillpad])
    dst = jnp.concatenate([edge_index[1].astype(jnp.int32), loop, fillpad])
    src3 = src.reshape(32, OUTER * INNER, CHUNK)
    dst3 = dst.reshape(32, OUTER * INNER, CHUNK)
    zvec = jnp.zeros((N_PAD,), _f32)

    degp = _deg_call(dst3)
    svec, y1p = _dense1_call(degp.reshape(2, 784, 128),
                             xp.T.reshape(2, 784, 128))
    y1f = y1p.reshape(2, N_PAD)
    aggp1 = _agg_call(y1f[0], y1f[1], zvec, src3, dst3)

    y2p = _dense2_call(aggp1.reshape(2, 2, 784, 128), svec,
                       W1.astype(_f32), b1.astype(_f32), W2.astype(_f32))
    y2f = y2p.reshape(2, N_PAD)
    aggp2 = _agg_call(y2f[0], y2f[1], zvec, src3, dst3)

    b2b = jnp.broadcast_to(b2.astype(_f32).reshape(2, 1, 1), (2, 1, 128))
    out = _fin_call(aggp2.reshape(2, 2, 784, 128),
                    svec, b2b)
    return jnp.stack([out[0].reshape(N_PAD), out[1].reshape(N_PAD)],
                     axis=1)[:N]
